# baseline (speedup 1.0000x reference)
import functools

import jax
import jax.numpy as jnp
from jax import lax
from jax.experimental import pallas as pl
from jax.experimental.pallas import tpu as pltpu
from jax.experimental.pallas import tpu_sc as plsc

N_ROWS = 16384
N_COLS = 26
EMB = 64
TOTAL = N_ROWS * N_COLS
NUM_CORES = 2
NUM_SUBCORES = 16
NW = NUM_CORES * NUM_SUBCORES
PER_W = TOTAL // NW
ROWS_W = N_ROWS // NW
CHUNK = 128
NCHUNK = PER_W // CHUNK
NBUF = 8
LAG = 4
LANES = 16
VPC = CHUNK // LANES
MAGIC = 20165
SHIFT = 19


@jax.jit
def _gather_sc(idx, table):
    mesh = plsc.VectorSubcoreMesh(
        core_axis_name="c", subcore_axis_name="s",
        num_cores=NUM_CORES, num_subcores=NUM_SUBCORES)

    @functools.partial(
        pl.kernel,
        mesh=mesh,
        out_type=jax.ShapeDtypeStruct((TOTAL, EMB), jnp.float32),
        scratch_types=[
            pltpu.VMEM((CHUNK, CHUNK), jnp.int32),
            pltpu.VMEM((NCHUNK, CHUNK), jnp.int32),
            pltpu.VMEM((NBUF, CHUNK, EMB), jnp.float32),
            pltpu.SemaphoreType.DMA((NBUF,)),
            pltpu.SemaphoreType.DMA((NBUF,)),
        ],
        compiler_params=pltpu.CompilerParams(
            use_tc_tiling_on_sc=False, needs_layout_passes=False),
    )
    def k(idx_hbm, table_hbm, out_hbm, idx_v, idx_f, rows_v, gsem, wsem):
        wid = lax.axis_index("s") * NUM_CORES + lax.axis_index("c")
        base = wid * PER_W
        pltpu.sync_copy(idx_hbm.at[pl.ds(wid * CHUNK, CHUNK)], idx_v)

        lane = lax.iota(jnp.int32, LANES)

        @pl.loop(0, NCHUNK)
        def _regroup(j):
            for v in range(VPC):
                p = j * CHUNK + v * LANES + lane
                q = p + 6 * lax.shift_right_logical(p * MAGIC, SHIFT)
                row = lax.shift_right_logical(q, 7)
                col = lax.bitwise_and(q, 127)
                idx_f[j, pl.ds(v * LANES, LANES)] = plsc.load_gather(
                    idx_v, [row, col])

        NTOT = NCHUNK + NBUF

        @pl.loop(0, NTOT, step=NBUF)
        def _steps(j0):
            for b in range(NBUF):
                j = j0 + b

                @pl.when(j < NCHUNK)
                def _gather_stage():
                    @pl.when(j >= NBUF)
                    def _reuse_wait():
                        pltpu.make_async_copy(
                            rows_v.at[b],
                            out_hbm.at[pl.ds(base, CHUNK)],
                            wsem.at[b]).wait()
                    pltpu.async_copy(
                        table_hbm.at[idx_f.at[j]],
                        rows_v.at[b], gsem.at[b])

                jw = j - LAG
                bw = (b - LAG) % NBUF

                @pl.when(jnp.logical_and(jw >= 0, jw < NCHUNK))
                def _write_stage():
                    pltpu.make_async_copy(
                        table_hbm.at[idx_f.at[0]],
                        rows_v.at[bw], gsem.at[bw]).wait()
                    pltpu.async_copy(
                        rows_v.at[bw],
                        out_hbm.at[pl.ds(base + jw * CHUNK, CHUNK)],
                        wsem.at[bw])

        for b in range(NBUF):
            pltpu.make_async_copy(
                rows_v.at[b], out_hbm.at[pl.ds(base, CHUNK)],
                wsem.at[b]).wait()

    out = k(idx, table)
    return out.reshape(N_ROWS, N_COLS, EMB)


def kernel(idx, table):
    idxp = jnp.pad(idx.astype(jnp.int32), ((0, 0), (0, 6)))
    idxp = idxp.reshape(N_ROWS // 4, CHUNK)
    return _gather_sc(idxp, table)


# speedup vs baseline: 1.0007x; 1.0007x over previous
import functools

import jax
import jax.numpy as jnp
from jax import lax
from jax.experimental import pallas as pl
from jax.experimental.pallas import tpu as pltpu
from jax.experimental.pallas import tpu_sc as plsc

N_ROWS = 16384
N_COLS = 26
EMB = 64
TOTAL = N_ROWS * N_COLS
NUM_CORES = 2
NUM_SUBCORES = 16
NW = NUM_CORES * NUM_SUBCORES
PER_W = TOTAL // NW
ROWS_W = N_ROWS // NW
CHUNK = 128
NCHUNK = PER_W // CHUNK
NBUF = 8
LAG = 4
LANES = 16
VPC = CHUNK // LANES
MAGIC = 20165
SHIFT = 19


@jax.jit
def _gather_sc(idx, table):
    mesh = plsc.VectorSubcoreMesh(
        core_axis_name="c", subcore_axis_name="s",
        num_cores=NUM_CORES, num_subcores=NUM_SUBCORES)

    @functools.partial(
        pl.kernel,
        mesh=mesh,
        out_type=jax.ShapeDtypeStruct((TOTAL, EMB), jnp.float32),
        scratch_types=[
            pltpu.VMEM((ROWS_W, 32), jnp.int32),
            pltpu.VMEM((NCHUNK, CHUNK), jnp.int32),
            pltpu.VMEM((NBUF, CHUNK, EMB), jnp.float32),
            pltpu.SemaphoreType.DMA((NBUF,)),
            pltpu.SemaphoreType.DMA((NBUF,)),
        ],
        compiler_params=pltpu.CompilerParams(
            use_tc_tiling_on_sc=False, needs_layout_passes=False),
    )
    def k(idx_hbm, table_hbm, out_hbm, idx_v, idx_f, rows_v, gsem, wsem):
        wid = lax.axis_index("s") * NUM_CORES + lax.axis_index("c")
        base = wid * PER_W
        pltpu.sync_copy(idx_hbm.at[pl.ds(wid * ROWS_W, ROWS_W)], idx_v)

        lane = lax.iota(jnp.int32, LANES)

        @pl.loop(0, NCHUNK)
        def _regroup(j):
            for v in range(VPC):
                p = j * CHUNK + v * LANES + lane
                row = lax.shift_right_logical(p * MAGIC, SHIFT)
                col = p - row * N_COLS
                idx_f[j, pl.ds(v * LANES, LANES)] = plsc.load_gather(
                    idx_v, [row, col])

        NTOT = NCHUNK + NBUF

        @pl.loop(0, NTOT, step=NBUF)
        def _steps(j0):
            for b in range(NBUF):
                j = j0 + b

                @pl.when(j < NCHUNK)
                def _gather_stage():
                    @pl.when(j >= NBUF)
                    def _reuse_wait():
                        pltpu.make_async_copy(
                            rows_v.at[b],
                            out_hbm.at[pl.ds(base, CHUNK)],
                            wsem.at[b]).wait()
                    pltpu.async_copy(
                        table_hbm.at[idx_f.at[j]],
                        rows_v.at[b], gsem.at[b])

                jw = j - LAG
                bw = (b - LAG) % NBUF

                @pl.when(jnp.logical_and(jw >= 0, jw < NCHUNK))
                def _write_stage():
                    pltpu.make_async_copy(
                        table_hbm.at[idx_f.at[0]],
                        rows_v.at[bw], gsem.at[bw]).wait()
                    pltpu.async_copy(
                        rows_v.at[bw],
                        out_hbm.at[pl.ds(base + jw * CHUNK, CHUNK)],
                        wsem.at[bw])

        for b in range(NBUF):
            pltpu.make_async_copy(
                rows_v.at[b], out_hbm.at[pl.ds(base, CHUNK)],
                wsem.at[b]).wait()

    out = k(idx, table)
    return out.reshape(N_ROWS, N_COLS, EMB)


def kernel(idx, table):
    idxp = jnp.pad(idx.astype(jnp.int32), ((0, 0), (0, 6)))
    return _gather_sc(idxp, table)


# speedup vs baseline: 1.2220x; 1.2212x over previous
import functools

import jax
import jax.numpy as jnp
from jax import lax
from jax.experimental import pallas as pl
from jax.experimental.pallas import tpu as pltpu
from jax.experimental.pallas import tpu_sc as plsc

N_ROWS = 16384
N_COLS = 26
EMB = 64
TOTAL = N_ROWS * N_COLS
NUM_CORES = 2
NUM_SUBCORES = 16
NW = NUM_CORES * NUM_SUBCORES
PER_W = TOTAL // NW
ROWS_W = N_ROWS // NW
GROUP = 4
NIDX = GROUP * N_COLS
NPAD = 112
NCHUNK = ROWS_W // GROUP
NBUF = 8
LAG = 4
LANES = 16
VPC = NPAD // LANES
MAGIC = 20165
SHIFT = 19


@jax.jit
def _gather_sc(idx, table):
    mesh = plsc.VectorSubcoreMesh(
        core_axis_name="c", subcore_axis_name="s",
        num_cores=NUM_CORES, num_subcores=NUM_SUBCORES)

    @functools.partial(
        pl.kernel,
        mesh=mesh,
        out_type=jax.ShapeDtypeStruct((N_ROWS, 32, 128), jnp.float32),
        scratch_types=[
            pltpu.VMEM((ROWS_W, 32), jnp.int32),
            pltpu.VMEM((NCHUNK, NPAD), jnp.int32),
            pltpu.VMEM((NBUF, NPAD, EMB), jnp.float32),
            pltpu.SemaphoreType.DMA((NBUF,)),
            pltpu.SemaphoreType.DMA((NBUF,)),
        ],
        compiler_params=pltpu.CompilerParams(
            use_tc_tiling_on_sc=False, needs_layout_passes=False),
    )
    def k(idx_hbm, table_hbm, out_hbm, idx_v, idx_f, rows_v, gsem, wsem):
        wid = lax.axis_index("s") * NUM_CORES + lax.axis_index("c")
        base = wid * ROWS_W
        pltpu.sync_copy(idx_hbm.at[pl.ds(base, ROWS_W)], idx_v)

        lane = lax.iota(jnp.int32, LANES)

        @pl.loop(0, NCHUNK)
        def _regroup(j):
            for v in range(VPC):
                p = j * NIDX + v * LANES + lane
                p = lax.min(p, PER_W - 1)
                row = lax.shift_right_logical(p * MAGIC, SHIFT)
                col = p - row * N_COLS
                idx_f[j, pl.ds(v * LANES, LANES)] = plsc.load_gather(
                    idx_v, [row, col])

        def wait_writes(b):
            for g in range(GROUP):
                pltpu.make_async_copy(
                    rows_v.at[b, pl.ds(g * N_COLS, N_COLS)],
                    out_hbm.at[base, pl.ds(0, N_COLS), pl.ds(0, EMB)],
                    wsem.at[b]).wait()

        NTOT = NCHUNK + NBUF

        @pl.loop(0, NTOT, step=NBUF)
        def _steps(j0):
            for b in range(NBUF):
                j = j0 + b

                @pl.when(j < NCHUNK)
                def _gather_stage():
                    @pl.when(j >= NBUF)
                    def _reuse_wait():
                        wait_writes(b)
                    pltpu.async_copy(
                        table_hbm.at[idx_f.at[j]],
                        rows_v.at[b], gsem.at[b])

                jw = j - LAG
                bw = (b - LAG) % NBUF

                @pl.when(jnp.logical_and(jw >= 0, jw < NCHUNK))
                def _write_stage():
                    pltpu.make_async_copy(
                        table_hbm.at[idx_f.at[0]],
                        rows_v.at[bw], gsem.at[bw]).wait()
                    for g in range(GROUP):
                        pltpu.async_copy(
                            rows_v.at[bw, pl.ds(g * N_COLS, N_COLS)],
                            out_hbm.at[base + jw * GROUP + g,
                                       pl.ds(0, N_COLS), pl.ds(0, EMB)],
                            wsem.at[bw])

        for b in range(NBUF):
            wait_writes(b)

    outp = k(idx, table)
    return outp[:, :N_COLS, :EMB]


def kernel(idx, table):
    idxp = jnp.pad(idx.astype(jnp.int32), ((0, 0), (0, 6)))
    return _gather_sc(idxp, table)
